# Initial kernel scaffold; baseline (speedup 1.0000x reference)
import functools

import jax
import jax.numpy as jnp
from jax import lax
from jax.experimental import pallas as pl
from jax.experimental.pallas import tpu as pltpu
from jax.experimental.pallas import tpu_sc as plsc

_L = 16


def _make_sc_gather_score(B, D, NNEG):
    NW = 32
    BW = B // NW
    CB = 128
    NCH = BW // CB
    KD = D // _L

    mesh = plsc.VectorSubcoreMesh(
        core_axis_name="c", subcore_axis_name="s", num_cores=2, num_subcores=16
    )

    def compute_chunk(ub, vb, nb, pb):
        def body(i, carry):
            pos = None
            neg = None
            for k in range(KD):
                sl = pl.ds(k * _L, _L)
                u = ub[i, sl]
                p = u * vb[i, sl]
                ns = nb[0, i, sl]
                for j in range(1, NNEG):
                    ns = ns + nb[j, i, sl]
                n = u * ns
                pos = p if pos is None else pos + p
                neg = n if neg is None else neg + n
            pb[i, pl.ds(0, _L)] = pos
            pb[i, pl.ds(_L, _L)] = neg
            return carry

        lax.fori_loop(0, CB, body, 0, unroll=2)

    @functools.partial(
        pl.kernel,
        out_type=jax.ShapeDtypeStruct((B, 2 * _L), jnp.float32),
        mesh=mesh,
        scratch_types=[
            pltpu.VMEM((BW,), jnp.int32),
            pltpu.VMEM((BW,), jnp.int32),
            pltpu.VMEM((BW * NNEG,), jnp.int32),
            pltpu.VMEM((2, CB, D), jnp.float32),
            pltpu.VMEM((2, CB, D), jnp.float32),
            pltpu.VMEM((2, NNEG, CB, D), jnp.float32),
            pltpu.VMEM((2, CB, 2 * _L), jnp.float32),
            pltpu.SemaphoreType.DMA,
            pltpu.SemaphoreType.DMA,
        ],
    )
    def sc_fn(u_pos_h, v_pos_h, vneg_h, u_tab, v_tab, out_h,
              u_idx, v_idx, n_idx, u_rows, v_rows, n_rows, pbuf, sem0, sem1):
        wid = lax.axis_index("c") * 16 + lax.axis_index("s")
        base = wid * BW
        pltpu.sync_copy(u_pos_h.at[pl.ds(base, BW)], u_idx)
        pltpu.sync_copy(v_pos_h.at[pl.ds(base, BW)], v_idx)
        pltpu.sync_copy(vneg_h.at[pl.ds(base * NNEG, BW * NNEG)], n_idx)

        sems = (sem0, sem1)

        def fire(c):
            p = c % 2
            s = sems[p]
            hs = [
                pltpu.async_copy(
                    u_tab.at[u_idx.at[pl.ds(c * CB, CB)]], u_rows.at[p], s),
                pltpu.async_copy(
                    v_tab.at[v_idx.at[pl.ds(c * CB, CB)]], v_rows.at[p], s),
            ]
            for j in range(NNEG):
                hs.append(pltpu.async_copy(
                    v_tab.at[n_idx.at[pl.ds((c * NNEG + j) * CB, CB)]],
                    n_rows.at[p, j], s))
            return hs

        pending = {0: fire(0)}
        for c in range(NCH):
            p = c % 2
            if c + 1 < NCH:
                pending[c + 1] = fire(c + 1)
            for h in pending.pop(c):
                h.wait()
            compute_chunk(u_rows.at[p], v_rows.at[p], n_rows.at[p], pbuf.at[p])
            pltpu.sync_copy(pbuf.at[p], out_h.at[pl.ds(base + c * CB, CB)])

    return sc_fn


def _finish(part, B):
    def body(x_ref, o_ref):
        x = x_ref[...]
        pos = jnp.sum(x[:, :_L], axis=1)
        neg = jnp.sum(x[:, _L:], axis=1)
        tot = jax.nn.log_sigmoid(pos) + jax.nn.log_sigmoid(-neg)
        o_ref[0, 0] = -jnp.sum(tot) / B

    return pl.pallas_call(
        body,
        out_shape=jax.ShapeDtypeStruct((1, 1), jnp.float32),
        in_specs=[pl.BlockSpec(memory_space=pltpu.VMEM)],
        out_specs=pl.BlockSpec(memory_space=pltpu.SMEM),
    )(part)


def kernel(u_pos, v_pos, v_neg, batch_size, U, V):
    B = u_pos.shape[0]
    D = U.shape[1]
    NNEG = v_neg.shape[1]
    vneg_flat = v_neg.reshape(B * NNEG)
    sc_fn = _make_sc_gather_score(B, D, NNEG)
    part = sc_fn(u_pos, v_pos, vneg_flat, U, V)
    out = _finish(part, B)
    return out[0, 0]


# speedup vs baseline: 1.7249x; 1.7249x over previous
import functools

import jax
import jax.numpy as jnp
from jax import lax
from jax.experimental import pallas as pl
from jax.experimental.pallas import tpu as pltpu
from jax.experimental.pallas import tpu_sc as plsc

_L = 16


def _make_sc_gather_score(B, D, NNEG):
    NW = 32
    BW = B // NW
    CB = 128
    NCH = BW // CB
    KD = D // _L

    mesh = plsc.VectorSubcoreMesh(
        core_axis_name="c", subcore_axis_name="s", num_cores=2, num_subcores=16
    )

    def compute_chunk(ub, vb, nb, pb):
        def body(i, carry):
            pos = None
            neg = None
            for k in range(KD):
                sl = pl.ds(k * _L, _L)
                u = ub[i, sl]
                p = u * vb[i, sl]
                ns = nb[0, i, sl]
                for j in range(1, NNEG):
                    ns = ns + nb[j, i, sl]
                n = u * ns
                pos = p if pos is None else pos + p
                neg = n if neg is None else neg + n
            pb[i, pl.ds(0, _L)] = pos
            pb[i, pl.ds(_L, _L)] = neg
            return carry

        lax.fori_loop(0, CB, body, 0, unroll=2)

    @functools.partial(
        pl.kernel,
        out_type=jax.ShapeDtypeStruct((B, 2 * _L), jnp.float32),
        mesh=mesh,
        scratch_types=[
            pltpu.VMEM((BW,), jnp.int32),
            pltpu.VMEM((BW,), jnp.int32),
            pltpu.VMEM((BW * NNEG,), jnp.int32),
            pltpu.VMEM((2, CB, D), jnp.float32),
            pltpu.VMEM((2, CB, D), jnp.float32),
            pltpu.VMEM((2, NNEG, CB, D), jnp.float32),
            pltpu.VMEM((2, CB, 2 * _L), jnp.float32),
            pltpu.SemaphoreType.DMA,
            pltpu.SemaphoreType.DMA,
        ],
        compiler_params=pltpu.CompilerParams(use_tc_tiling_on_sc=False),
    )
    def sc_fn(u_pos_h, v_pos_h, vneg_h, u_tab, v_tab, out_h,
              u_idx, v_idx, n_idx, u_rows, v_rows, n_rows, pbuf, sem0, sem1):
        wid = lax.axis_index("c") * 16 + lax.axis_index("s")
        base = wid * BW
        pltpu.sync_copy(u_pos_h.at[pl.ds(base, BW)], u_idx)
        pltpu.sync_copy(v_pos_h.at[pl.ds(base, BW)], v_idx)
        pltpu.sync_copy(vneg_h.at[pl.ds(base * NNEG, BW * NNEG)], n_idx)

        sems = (sem0, sem1)

        def fire(c):
            p = c % 2
            s = sems[p]
            hs = [
                pltpu.async_copy(
                    u_tab.at[u_idx.at[pl.ds(c * CB, CB)]], u_rows.at[p], s),
                pltpu.async_copy(
                    v_tab.at[v_idx.at[pl.ds(c * CB, CB)]], v_rows.at[p], s),
            ]
            for j in range(NNEG):
                hs.append(pltpu.async_copy(
                    v_tab.at[n_idx.at[pl.ds((c * NNEG + j) * CB, CB)]],
                    n_rows.at[p, j], s))
            return hs

        pending = {0: fire(0)}
        for c in range(NCH):
            p = c % 2
            if c + 1 < NCH:
                pending[c + 1] = fire(c + 1)
            for h in pending.pop(c):
                h.wait()
            compute_chunk(u_rows.at[p], v_rows.at[p], n_rows.at[p], pbuf.at[p])
            pltpu.sync_copy(pbuf.at[p], out_h.at[pl.ds(base + c * CB, CB)])

    return sc_fn


def _finish(part, B):
    def body(x_ref, o_ref):
        x = x_ref[...]
        pos = jnp.sum(x[:, :_L], axis=1)
        neg = jnp.sum(x[:, _L:], axis=1)
        tot = jax.nn.log_sigmoid(pos) + jax.nn.log_sigmoid(-neg)
        o_ref[0, 0] = -jnp.sum(tot) / B

    return pl.pallas_call(
        body,
        out_shape=jax.ShapeDtypeStruct((1, 1), jnp.float32),
        in_specs=[pl.BlockSpec(memory_space=pltpu.VMEM)],
        out_specs=pl.BlockSpec(memory_space=pltpu.SMEM),
    )(part)


def kernel(u_pos, v_pos, v_neg, batch_size, U, V):
    B = u_pos.shape[0]
    D = U.shape[1]
    NNEG = v_neg.shape[1]
    vneg_flat = v_neg.reshape(B * NNEG)
    sc_fn = _make_sc_gather_score(B, D, NNEG)
    part = sc_fn(u_pos, v_pos, vneg_flat, U, V)
    out = _finish(part, B)
    return out[0, 0]
